# baseline (speedup 1.0000x reference)
import functools

import jax
import jax.numpy as jnp
from jax import lax
from jax.experimental import pallas as pl
from jax.experimental.pallas import tpu as pltpu
from jax.experimental.pallas import tpu_sc as plsc

B = 4096
HIST = 200
D = 64

_NC, _NS = 2, 16
_NW = _NC * _NS
_CHUNK = 128


def _sc_gather_body(qidx, tidx, qtab, ttab, qout, tout, idx_v, rows_v, sem):
    wid = lax.axis_index("s") * _NC + lax.axis_index("c")
    nq = B * HIST // _NW
    nt = B * D // _NW
    qbase = wid * nq
    tbase = wid * nt

    def q_step(c, carry):
        row = qbase + c * _CHUNK
        pltpu.sync_copy(qidx.at[pl.ds(row, _CHUNK)], idx_v)
        pltpu.async_copy(qtab.at[idx_v], rows_v, sem).wait()
        pltpu.sync_copy(rows_v, qout.at[pl.ds(row, _CHUNK)])
        return carry

    lax.fori_loop(0, nq // _CHUNK, q_step, 0)

    def t_step(c, carry):
        row = tbase + c * _CHUNK
        pltpu.sync_copy(tidx.at[pl.ds(row, _CHUNK)], idx_v)
        pltpu.async_copy(ttab.at[idx_v], rows_v, sem).wait()
        pltpu.sync_copy(rows_v, tout.at[pl.ds(row, _CHUNK)])
        return carry

    lax.fori_loop(0, nt // _CHUNK, t_step, 0)


def _sc_gather(qidx, tidx, qtab, ttab):
    mesh = plsc.VectorSubcoreMesh(core_axis_name="c", subcore_axis_name="s")
    return pl.kernel(
        _sc_gather_body,
        out_type=(
            jax.ShapeDtypeStruct((B * HIST, D), jnp.float32),
            jax.ShapeDtypeStruct((B * D, D), jnp.float32),
        ),
        mesh=mesh,
        compiler_params=pltpu.CompilerParams(use_tc_tiling_on_sc=False),
        scratch_types=[
            pltpu.VMEM((_CHUNK,), jnp.int32),
            pltpu.VMEM((_CHUNK, D), jnp.float32),
            pltpu.SemaphoreType.DMA,
        ],
    )(qidx, tidx, qtab, ttab)


_G = 8


def _bmm_body(q_ref, t_ref, o_ref):
    for i in range(_G):
        o_ref[i] = jnp.dot(q_ref[i], t_ref[i],
                           preferred_element_type=jnp.float32)


def _tc_bmm(q3, t3):
    return pl.pallas_call(
        _bmm_body,
        grid=(B // _G,),
        in_specs=[
            pl.BlockSpec((_G, HIST, D), lambda g: (g, 0, 0)),
            pl.BlockSpec((_G, D, D), lambda g: (g, 0, 0)),
        ],
        out_specs=pl.BlockSpec((_G, HIST, D), lambda g: (g, 0, 0)),
        out_shape=jax.ShapeDtypeStruct((B, HIST, D), jnp.float32),
    )(q3, t3)


def kernel(user, item_i, item_j, user_bias_table, item_bias_table,
           query_table, target_table):
    qidx = item_j.reshape(-1)
    tidx = item_i.reshape(-1)
    q_gath, t_gath = _sc_gather(qidx, tidx, query_table, target_table)
    return _tc_bmm(q_gath.reshape(B, HIST, D), t_gath.reshape(B, D, D))


# speedup vs baseline: 1.1240x; 1.1240x over previous
import functools

import jax
import jax.numpy as jnp
from jax import lax
from jax.experimental import pallas as pl
from jax.experimental.pallas import tpu as pltpu
from jax.experimental.pallas import tpu_sc as plsc

B = 4096
HIST = 200
D = 64

_NC, _NS = 2, 16
_NW = _NC * _NS
_CH = 128
_NB = 4
_SETS = 2
_SG = _CH * _NB * _SETS

_NQ = B * HIST // _NW
_NT = B * D // _NW


def _gather_stream(tab, idx_v, out, row0, chunk0, nsuper, bufs, gsem, wsem):
    def body(g, carry):
        base = g * _SG
        for s in range(_SETS):
            sbase = base + s * _NB * _CH

            @pl.when(g > 0)
            def _():
                for b in range(_NB):
                    pltpu.make_async_copy(
                        bufs.at[s].at[b],
                        out.at[pl.ds(row0, _CH)],
                        wsem.at[s],
                    ).wait()

            handles = []
            for b in range(_NB):
                lc = chunk0 + g * (_SETS * _NB) + s * _NB + b
                h = pltpu.make_async_copy(
                    tab.at[idx_v.at[lc]], bufs.at[s].at[b], gsem.at[s])
                h.start()
                handles.append(h)
            for h in handles:
                h.wait()
            for b in range(_NB):
                crow = row0 + sbase + b * _CH
                pltpu.make_async_copy(
                    bufs.at[s].at[b], out.at[pl.ds(crow, _CH)], wsem.at[s]
                ).start()
        return carry

    lax.fori_loop(0, nsuper, body, 0)
    for s in range(_SETS):
        for b in range(_NB):
            pltpu.make_async_copy(
                bufs.at[s].at[b], out.at[pl.ds(row0, _CH)], wsem.at[s]
            ).wait()


def _sc_gather_body(qidx, tidx, qtab, ttab, qout, tout,
                    idx_v, bufs, gsem, wsem):
    wid = lax.axis_index("s") * _NC + lax.axis_index("c")
    nqc = _NQ // _CH
    ntc = _NT // _CH
    pltpu.sync_copy(qidx.at[pl.ds(wid * nqc, nqc)], idx_v.at[pl.ds(0, nqc)])
    pltpu.sync_copy(tidx.at[pl.ds(wid * ntc, ntc)],
                    idx_v.at[pl.ds(nqc, ntc)])
    _gather_stream(qtab, idx_v, qout, wid * _NQ, 0, _NQ // _SG,
                   bufs, gsem, wsem)
    _gather_stream(ttab, idx_v, tout, wid * _NT, nqc, _NT // _SG,
                   bufs, gsem, wsem)


def _sc_gather(qidx2d, tidx2d, qtab, ttab):
    mesh = plsc.VectorSubcoreMesh(core_axis_name="c", subcore_axis_name="s")
    return pl.kernel(
        _sc_gather_body,
        out_type=(
            jax.ShapeDtypeStruct((B * HIST, D), jnp.float32),
            jax.ShapeDtypeStruct((B * D, D), jnp.float32),
        ),
        mesh=mesh,
        compiler_params=pltpu.CompilerParams(use_tc_tiling_on_sc=False),
        scratch_types=[
            pltpu.VMEM(((_NQ + _NT) // _CH, _CH), jnp.int32),
            pltpu.VMEM((_SETS, _NB, _CH, D), jnp.float32),
            pltpu.SemaphoreType.DMA((_SETS,)),
            pltpu.SemaphoreType.DMA((_SETS,)),
        ],
    )(qidx2d, tidx2d, qtab, ttab)


_G = 8


def _bmm_body(q_ref, t_ref, o_ref):
    for i in range(_G):
        o_ref[i] = jnp.dot(q_ref[i], t_ref[i],
                           preferred_element_type=jnp.float32)


def _tc_bmm(q3, t3):
    return pl.pallas_call(
        _bmm_body,
        grid=(B // _G,),
        in_specs=[
            pl.BlockSpec((_G, HIST, D), lambda g: (g, 0, 0)),
            pl.BlockSpec((_G, D, D), lambda g: (g, 0, 0)),
        ],
        out_specs=pl.BlockSpec((_G, HIST, D), lambda g: (g, 0, 0)),
        out_shape=jax.ShapeDtypeStruct((B, HIST, D), jnp.float32),
    )(q3, t3)


def kernel(user, item_i, item_j, user_bias_table, item_bias_table,
           query_table, target_table):
    qidx = item_j.reshape(-1, _CH)
    tidx = item_i.reshape(-1, _CH)
    q_gath, t_gath = _sc_gather(qidx, tidx, query_table, target_table)
    return _tc_bmm(q_gath.reshape(B, HIST, D), t_gath.reshape(B, D, D))
